# baseline (speedup 1.0000x reference)
import functools

import jax
import jax.numpy as jnp
from jax import lax
from jax.experimental import pallas as pl
from jax.experimental.pallas import tpu as pltpu
from jax.experimental.pallas import tpu_sc as plsc

B, L, V, D = 1024, 200, 1000000, 64
NROWS = B * L
NW = 32
RW = NROWS // NW
CHUNK = 640
SUB = 128
NSUB = CHUNK // SUB
NCHUNK = RW // CHUNK


def _sc_gather(seq_flat, token_table):
    mesh = plsc.VectorSubcoreMesh(core_axis_name="c", subcore_axis_name="s")

    @functools.partial(
        pl.kernel,
        out_type=jax.ShapeDtypeStruct((NROWS, D), jnp.float32),
        mesh=mesh,
        scratch_types=[
            pltpu.VMEM((CHUNK,), jnp.int32),
            pltpu.VMEM((CHUNK, D), jnp.float32),
            pltpu.SemaphoreType.DMA,
        ],
        compiler_params=pltpu.CompilerParams(use_tc_tiling_on_sc=False),
    )
    def gather_kernel(idx_hbm, table_hbm, out_hbm, idx_v, rows_v, sem):
        wid = lax.axis_index("s") * 2 + lax.axis_index("c")
        base = wid * RW

        def chunk_body(g, carry):
            off = base + g * CHUNK
            pltpu.sync_copy(idx_hbm.at[pl.ds(off, CHUNK)], idx_v)
            cps = []
            for j in range(NSUB):
                cps.append(
                    pltpu.async_copy(
                        table_hbm.at[idx_v.at[pl.ds(j * SUB, SUB)]],
                        rows_v.at[pl.ds(j * SUB, SUB)],
                        sem,
                    )
                )
            for cp in cps:
                cp.wait()
            pltpu.sync_copy(rows_v, out_hbm.at[pl.ds(off, CHUNK)])
            return carry

        lax.fori_loop(0, NCHUNK, chunk_body, 0)

    return gather_kernel(seq_flat, token_table)


BB1 = 64
RB1 = BB1 * L


def _tc_out1(g2, ph2, pos_table, unit_embed):
    def body(g_ref, ph_ref, pos_ref, ue_ref, out_ref):
        unit = lax.dot_general(
            ph_ref[...], ue_ref[...],
            (((1,), (0,)), ((), ())),
            preferred_element_type=jnp.float32,
        )
        posb = jnp.broadcast_to(pos_ref[...][None], (BB1, L, D)).reshape(RB1, D)
        out_ref[...] = g_ref[...] + unit + posb

    return pl.pallas_call(
        body,
        grid=(B // BB1,),
        in_specs=[
            pl.BlockSpec((RB1, D), lambda i: (i, 0)),
            pl.BlockSpec((RB1, 7), lambda i: (i, 0)),
            pl.BlockSpec((L, D), lambda i: (0, 0)),
            pl.BlockSpec((7, D), lambda i: (0, 0)),
        ],
        out_specs=pl.BlockSpec((RB1, D), lambda i: (i, 0)),
        out_shape=jax.ShapeDtypeStruct((NROWS, D), jnp.float32),
    )(g2, ph2, pos_table, unit_embed)


BB2 = 64


def _tc_out2(meta_info, padding, case_embed):
    def body(meta_ref, pad_ref, case_ref, out_ref):
        m = meta_ref[...]
        p = pad_ref[...]
        prod = m[:, None, :] * p[None, :, :]
        res = lax.dot_general(
            prod.reshape(BB2 * L, D), case_ref[...],
            (((1,), (0,)), ((), ())),
            preferred_element_type=jnp.float32,
        )
        out_ref[...] = res.reshape(BB2, L, D)

    return pl.pallas_call(
        body,
        grid=(B // BB2,),
        in_specs=[
            pl.BlockSpec((BB2, D), lambda i: (i, 0)),
            pl.BlockSpec((L, D), lambda i: (0, 0)),
            pl.BlockSpec((D, D), lambda i: (0, 0)),
        ],
        out_specs=pl.BlockSpec((BB2, L, D), lambda i: (i, 0, 0)),
        out_shape=jax.ShapeDtypeStruct((B, L, D), jnp.float32),
    )(meta_info, padding, case_embed)


def kernel(sequence, meta_info, ph_dimensions, token_table, pos_table,
           case_embed, unit_embed, padding):
    seq_flat = sequence.reshape(NROWS).astype(jnp.int32)
    g2 = _sc_gather(seq_flat, token_table)
    ph2 = ph_dimensions.astype(jnp.float32).reshape(NROWS, 7)
    out1 = _tc_out1(g2, ph2, pos_table, unit_embed).reshape(B, L, D)
    out2 = _tc_out2(meta_info, padding, case_embed)
    return (out1, out2)
